# baseline (speedup 1.0000x reference)
import functools

import jax
import jax.numpy as jnp
from jax import lax
from jax.experimental import pallas as pl
from jax.experimental.pallas import tpu as pltpu

N = 50000
E = 800000
HEADS = 4
HID = 16
OUT_DIM = 32
BN_SCALE = float(1.0 / (1.0 + 1e-5) ** 0.5)

R = 2000
NBLK = N // R


def _leaky(v):
    return jnp.where(v > 0, v, 0.2 * v)


def _stage_pre_body(acc_in, W_ref, as_ref, ad_ref, b_ref, g_ref, be_ref,
                    s_ref, d_ref, init_ref, *, first, heads, chid,
                    pheads, pchid):
    hw = heads * chid
    if first:
        t = acc_in[...]
    else:
        phw = pheads * pchid
        agg = acc_in[:, :phw].reshape(R, pheads, pchid)
        den = acc_in[:, phw:phw + pheads].reshape(R, pheads, 1)
        z = (agg / (den + 1e-16)).reshape(R, phw) + b_ref[...]
        t = jax.nn.relu(g_ref[...] * z * BN_SCALE + be_ref[...])
    h = jnp.dot(t, W_ref[...], preferred_element_type=jnp.float32)
    hr = h.reshape(R, heads, chid)
    es = (hr * as_ref[...][None]).sum(-1)
    ed = (hr * ad_ref[...][None]).sum(-1)
    spad = s_ref.shape[1] - hw - heads
    s_ref[...] = jnp.concatenate(
        [h, es, jnp.zeros((R, spad), jnp.float32)], axis=1)
    d_ref[...] = jnp.concatenate(
        [ed, jnp.zeros((R, 16 - heads), jnp.float32)], axis=1)
    w = jnp.exp(_leaky(es + ed))
    initagg = (hr * w[..., None]).reshape(R, hw)
    ipad = init_ref.shape[1] - hw - heads
    init_ref[...] = jnp.concatenate(
        [initagg, w, jnp.zeros((R, ipad), jnp.float32)], axis=1)


def _make_stage_pre(in_w, heads, chid, s_w, acc_w, first,
                    pheads=HEADS, pchid=HID):
    body = functools.partial(_stage_pre_body, first=first, heads=heads,
                             chid=chid, pheads=pheads, pchid=pchid)
    hw = heads * chid
    w_in = in_w if first else pheads * pchid
    full = lambda *shape: pl.BlockSpec(shape, lambda i: (0,) * len(shape))
    return pl.pallas_call(
        body,
        grid=(NBLK,),
        in_specs=[
            pl.BlockSpec((R, in_w), lambda i: (i, 0)),
            full(w_in, hw),
            full(heads, chid),
            full(heads, chid),
            full(pheads * pchid),
            full(pheads * pchid),
            full(pheads * pchid),
        ],
        out_specs=[
            pl.BlockSpec((R, s_w), lambda i: (i, 0)),
            pl.BlockSpec((R, 16), lambda i: (i, 0)),
            pl.BlockSpec((R, acc_w), lambda i: (i, 0)),
        ],
        out_shape=[
            jax.ShapeDtypeStruct((N, s_w), jnp.float32),
            jax.ShapeDtypeStruct((N, 16), jnp.float32),
            jax.ShapeDtypeStruct((N, acc_w), jnp.float32),
        ],
    )


def _mlp_body(acc_ref, b3_ref, c1_ref, Wm1b_ref, Wm2_ref, bm2_ref, Wm3_ref,
              bm3_ref, out_ref):
    emb = acc_ref[:, :OUT_DIM] / (acc_ref[:, OUT_DIM:OUT_DIM + 1] + 1e-16)
    emb = emb + b3_ref[...]
    z = jax.nn.relu(jnp.dot(emb, Wm1b_ref[...],
                            preferred_element_type=jnp.float32) + c1_ref[...])
    z = jax.nn.relu(jnp.dot(z, Wm2_ref[...],
                            preferred_element_type=jnp.float32) + bm2_ref[...])
    out_ref[...] = jax.nn.sigmoid(
        jnp.dot(z, Wm3_ref[...], preferred_element_type=jnp.float32)
        + bm3_ref[...])


def _make_mlp():
    full = lambda *shape: pl.BlockSpec(shape, lambda i: (0,) * len(shape))
    return pl.pallas_call(
        _mlp_body,
        grid=(NBLK,),
        in_specs=[
            pl.BlockSpec((R, 48), lambda i: (i, 0)),
            full(OUT_DIM),
            full(1, 64),
            full(OUT_DIM, 64),
            full(64, 32),
            full(32),
            full(32, 1),
            full(1),
        ],
        out_specs=pl.BlockSpec((R, 1), lambda i: (i, 0)),
        out_shape=jax.ShapeDtypeStruct((N, 1), jnp.float32),
    )


def _edge_pass_jnp(S, D, INIT, src, dst, heads, chid):
    hw = heads * chid
    es = S[:, hw:hw + heads]
    ed = D[:, :heads]
    w = jnp.exp(_leaky(es[src] + ed[dst]))
    hs = S[src, :hw].reshape(E, heads, chid)
    agg = jax.ops.segment_sum((hs * w[..., None]).reshape(E, hw), dst,
                              num_segments=N)
    den = jax.ops.segment_sum(w, dst, num_segments=N)
    acc = INIT.at[:, :hw].add(agg)
    acc = acc.at[:, hw:hw + heads].add(den)
    return acc


def kernel(x, edge_index, source_node, W1, a1s, a1d, b1, g1, be1, W2, a2s,
           a2d, b2, g2, be2, W3, a3s, a3d, b3, Wm1, bm1, Wm2, bm2, Wm3, bm3):
    src = edge_index[0].astype(jnp.int32)
    dst = edge_index[1].astype(jnp.int32)

    zeros64 = jnp.zeros((64,), jnp.float32)
    stage1 = _make_stage_pre(7, HEADS, HID, 80, 80, True)
    S1, D1, I1 = stage1(x, W1, a1s, a1d, zeros64, zeros64, zeros64)
    acc1 = _edge_pass_jnp(S1, D1, I1, src, dst, HEADS, HID)

    stage2 = _make_stage_pre(80, HEADS, HID, 80, 80, False)
    S2, D2, I2 = stage2(acc1, W2, a2s, a2d, b1, g1, be1)
    acc2 = _edge_pass_jnp(S2, D2, I2, src, dst, HEADS, HID)

    stage3 = _make_stage_pre(80, 1, OUT_DIM, 48, 48, False)
    S3, D3, I3 = stage3(acc2, W3, a3s, a3d, b2, g2, be2)
    acc3 = _edge_pass_jnp(S3, D3, I3, src, dst, 1, OUT_DIM)

    se = acc3[source_node, :OUT_DIM] / (acc3[source_node, OUT_DIM] + 1e-16)
    se = (se + b3)[None, :]
    c1 = se @ Wm1[:OUT_DIM] + bm1[None, :]
    out = _make_mlp()(acc3, b3, c1, Wm1[OUT_DIM:], Wm2, bm2, Wm3, bm3)
    return out[:, 0]


# speedup vs baseline: 34.9065x; 34.9065x over previous
import functools

import jax
import jax.numpy as jnp
from jax import lax
from jax.experimental import pallas as pl
from jax.experimental.pallas import tpu as pltpu
from jax.experimental.pallas import tpu_sc as plsc

N = 50000
E = 800000
HEADS = 4
HID = 16
OUT_DIM = 32
BN_SCALE = float(1.0 / (1.0 + 1e-5) ** 0.5)

R = 2000
NBLK = N // R


def _leaky(v):
    return jnp.where(v > 0, v, 0.2 * v)


_GDN = lax.GatherDimensionNumbers(offset_dims=(), collapsed_slice_dims=(0,),
                                  start_index_map=(0,))


def _vgather(v, idx):
    return lax.gather(v, idx[:, None], _GDN, (1,),
                      mode=lax.GatherScatterMode.PROMISE_IN_BOUNDS)


def _bcast_lane(v, lane_id):
    return _vgather(v, jnp.full((16,), lane_id, jnp.int32))


def _stage_pre_body(acc_in, den_in, W_ref, as_ref, ad_ref, b_ref, g_ref,
                    be_ref, s_ref, d_ref, ia_ref, id_ref, *, first, heads,
                    chid, pheads, pchid):
    hw = heads * chid
    if first:
        t = acc_in[...]
    else:
        phw = pheads * pchid
        agg = acc_in[...].reshape(R, pheads, pchid)
        den = den_in[...][:, :pheads].reshape(R, pheads, 1)
        z = (agg / (den + 1e-16)).reshape(R, phw) + b_ref[...]
        t = jax.nn.relu(g_ref[...] * z * BN_SCALE + be_ref[...])
    h = jnp.dot(t, W_ref[...], preferred_element_type=jnp.float32)
    hr = h.reshape(R, heads, chid)
    es = (hr * as_ref[...][None]).sum(-1)
    ed = (hr * ad_ref[...][None]).sum(-1)
    spad = s_ref.shape[1] - hw - heads
    s_ref[...] = jnp.concatenate(
        [h, es, jnp.zeros((R, spad), jnp.float32)], axis=1)
    d_ref[...] = jnp.concatenate(
        [ed, jnp.zeros((R, 16 - heads), jnp.float32)], axis=1)
    w = jnp.exp(_leaky(es + ed))
    ia_ref[...] = (hr * w[..., None]).reshape(R, hw)
    id_ref[...] = jnp.concatenate(
        [w, jnp.zeros((R, 4 - heads), jnp.float32)], axis=1) \
        if heads < 4 else w


def _make_stage_pre(in_w, heads, chid, s_w, first, pheads=HEADS, pchid=HID):
    body = functools.partial(_stage_pre_body, first=first, heads=heads,
                             chid=chid, pheads=pheads, pchid=pchid)
    hw = heads * chid
    w_in = in_w if first else pheads * pchid
    full = lambda *shape: pl.BlockSpec(shape, lambda i: (0,) * len(shape))
    return pl.pallas_call(
        body,
        grid=(NBLK,),
        in_specs=[
            pl.BlockSpec((R, in_w), lambda i: (i, 0)),
            pl.BlockSpec((R, 4), lambda i: (i, 0)),
            full(w_in, hw),
            full(heads, chid),
            full(heads, chid),
            full(pheads * pchid),
            full(pheads * pchid),
            full(pheads * pchid),
        ],
        out_specs=[
            pl.BlockSpec((R, s_w), lambda i: (i, 0)),
            pl.BlockSpec((R, 16), lambda i: (i, 0)),
            pl.BlockSpec((R, hw), lambda i: (i, 0)),
            pl.BlockSpec((R, 4), lambda i: (i, 0)),
        ],
        out_shape=[
            jax.ShapeDtypeStruct((N, s_w), jnp.float32),
            jax.ShapeDtypeStruct((N, 16), jnp.float32),
            jax.ShapeDtypeStruct((N, hw), jnp.float32),
            jax.ShapeDtypeStruct((N, 4), jnp.float32),
        ],
    )


def _mlp_body(acc_ref, den_ref, b3_ref, c1_ref, Wm1b_ref, Wm2_ref, bm2_ref,
              Wm3_ref, bm3_ref, out_ref):
    den = den_ref[...][:, :1]
    emb = acc_ref[...] / (den + 1e-16) + b3_ref[...]
    z = jax.nn.relu(jnp.dot(emb, Wm1b_ref[...],
                            preferred_element_type=jnp.float32) + c1_ref[...])
    z = jax.nn.relu(jnp.dot(z, Wm2_ref[...],
                            preferred_element_type=jnp.float32) + bm2_ref[...])
    out_ref[...] = jax.nn.sigmoid(
        jnp.dot(z, Wm3_ref[...], preferred_element_type=jnp.float32)
        + bm3_ref[...])


def _make_mlp():
    full = lambda *shape: pl.BlockSpec(shape, lambda i: (0,) * len(shape))
    return pl.pallas_call(
        _mlp_body,
        grid=(NBLK,),
        in_specs=[
            pl.BlockSpec((R, OUT_DIM), lambda i: (i, 0)),
            pl.BlockSpec((R, 4), lambda i: (i, 0)),
            full(OUT_DIM),
            full(1, 64),
            full(OUT_DIM, 64),
            full(64, 32),
            full(32),
            full(32, 1),
            full(1),
        ],
        out_specs=pl.BlockSpec((R, 1), lambda i: (i, 0)),
        out_shape=jax.ShapeDtypeStruct((N, 1), jnp.float32),
    )


NH = N // 2
B = 128
NB = 391
CT = B * NB
EP = 16 * CT
ACH = NH // 8


def _make_edge_sc(heads, chid, s_w):
    hw = heads * chid
    mesh = plsc.VectorSubcoreMesh(core_axis_name="c", subcore_axis_name="s")

    @functools.partial(
        pl.kernel, mesh=mesh,
        compiler_params=pltpu.CompilerParams(use_tc_tiling_on_sc=False),
        out_type=[
            jax.ShapeDtypeStruct((N, hw), jnp.float32),
            jax.ShapeDtypeStruct((N // 4, 16), jnp.float32),
        ],
        scratch_types=[
            pltpu.VMEM((B,), jnp.int32),
            pltpu.VMEM((B,), jnp.int32),
            pltpu.VMEM((B,), jnp.int32),
            pltpu.VMEM((B,), jnp.int32),
            pltpu.VMEM((B,), jnp.int32),
            pltpu.VMEM((B,), jnp.int32),
            pltpu.VMEM((B, s_w), jnp.float32),
            pltpu.VMEM((B, 16), jnp.float32),
            pltpu.VMEM((B, hw), jnp.float32),
            pltpu.VMEM((B, 16), jnp.float32),
            pltpu.VMEM_SHARED((NH + 8, hw), jnp.float32),
            pltpu.VMEM_SHARED((NH // 4 + 8, 16), jnp.float32),
            pltpu.SemaphoreType.DMA,
            pltpu.SemaphoreType.DMA,
        ],
    )
    def edge_kernel(S, Dt, IA, ID, srcp, dstp, accout, denout,
                    sidx, didx, gd, lidx, l4, lq, srows, drows, rrows, dnrows,
                    acc, dacc, sem1, sem2):
        cid = lax.axis_index("c")
        sid = lax.axis_index("s")
        lane = lax.iota(jnp.int32, 16)
        lanem4 = lane & 3
        lanegrp = lane >> 2
        headm = lanem4 < heads

        @pl.when(sid < 8)
        def _init_acc():
            pltpu.sync_copy(IA.at[pl.ds(cid * NH + sid * ACH, ACH)],
                            acc.at[pl.ds(sid * ACH, ACH)])

        @pl.when(sid == 8)
        def _init_den():
            pltpu.sync_copy(ID.at[pl.ds(cid * (NH // 4), NH // 4)],
                            dacc.at[pl.ds(0, NH // 4)])

        plsc.subcore_barrier()

        def batch_body(b, carry):
            base = sid * CT + b * B
            pltpu.sync_copy(srcp.at[pl.ds(base, B)], sidx)
            pltpu.sync_copy(dstp.at[pl.ds(base, B)], didx)
            for j in range(B // 16):
                sl = pl.ds(j * 16, 16)
                dv = didx[sl]
                gd[sl] = jnp.minimum(dv, N - 1)
                lv = dv - cid * NH
                ok = (lv >= 0) & (lv < NH)
                lv = jnp.where(ok, lv, NH)
                lidx[sl] = lv
                l4[sl] = lv >> 2
                lq[sl] = lv & 3
            cp1 = pltpu.async_copy(S.at[sidx], srows, sem1)
            cp2 = pltpu.async_copy(Dt.at[gd], drows, sem2)
            cp1.wait()
            cp2.wait()

            def edge_body(e, ecarry):
                u = srows[e, pl.ds(hw, 16)] + drows[e, pl.ds(0, 16)]
                u = jnp.where(u > 0, u, 0.2 * u)
                w = jnp.exp(u)
                eb = (e >> 4) << 4
                q16 = lq[pl.ds(eb, 16)]
                qb = _vgather(q16, jnp.full((16,), 0, jnp.int32) + (e - eb))
                rep4 = _vgather(w, lanem4)
                dnrows[e, pl.ds(0, 16)] = jnp.where(
                    (lanegrp == qb) & headm, rep4, 0.0)
                for g in range(hw // 16):
                    hd = g // (chid // 16)
                    wb = _bcast_lane(w, hd)
                    rrows[e, pl.ds(g * 16, 16)] = (
                        wb * srows[e, pl.ds(g * 16, 16)])
                return ecarry

            lax.fori_loop(0, B, edge_body, 0)
            pltpu.sync_copy(rrows, acc.at[lidx], add=True)
            pltpu.sync_copy(dnrows, dacc.at[l4], add=True)
            return carry

        lax.fori_loop(0, NB, batch_body, 0)
        plsc.subcore_barrier()

        @pl.when(sid < 8)
        def _dump_acc():
            pltpu.sync_copy(acc.at[pl.ds(sid * ACH, ACH)],
                            accout.at[pl.ds(cid * NH + sid * ACH, ACH)])

        @pl.when(sid == 8)
        def _dump_den():
            pltpu.sync_copy(dacc.at[pl.ds(0, NH // 4)],
                            denout.at[pl.ds(cid * (NH // 4), NH // 4)])

    return edge_kernel


def kernel(x, edge_index, source_node, W1, a1s, a1d, b1, g1, be1, W2, a2s,
           a2d, b2, g2, be2, W3, a3s, a3d, b3, Wm1, bm1, Wm2, bm2, Wm3, bm3):
    src = edge_index[0].astype(jnp.int32)
    dst = edge_index[1].astype(jnp.int32)
    pad = EP - E
    srcp = jnp.concatenate([src, jnp.zeros((pad,), jnp.int32)])
    dstp = jnp.concatenate([dst, jnp.full((pad,), N, jnp.int32)])

    zeros64 = jnp.zeros((64,), jnp.float32)
    dummy_den = jnp.zeros((N, 4), jnp.float32)
    stage1 = _make_stage_pre(7, HEADS, HID, 80, True)
    S1, D1, IA1, ID1 = stage1(x, dummy_den, W1, a1s, a1d, zeros64, zeros64,
                              zeros64)
    edge44 = _make_edge_sc(HEADS, HID, 80)
    acc1, den1 = edge44(S1, D1, IA1, ID1.reshape(N // 4, 16), srcp, dstp)
    den1 = den1.reshape(N, 4)

    stage2 = _make_stage_pre(64, HEADS, HID, 80, False)
    S2, D2, IA2, ID2 = stage2(acc1, den1, W2, a2s, a2d, b1, g1, be1)
    acc2, den2 = edge44(S2, D2, IA2, ID2.reshape(N // 4, 16), srcp, dstp)
    den2 = den2.reshape(N, 4)

    stage3 = _make_stage_pre(64, 1, OUT_DIM, 48, False)
    S3, D3, IA3, ID3 = stage3(acc2, den2, W3, a3s, a3d, b2, g2, be2)
    edge13 = _make_edge_sc(1, OUT_DIM, 48)
    acc3, den3 = edge13(S3, D3, IA3, ID3.reshape(N // 4, 16), srcp, dstp)
    den3 = den3.reshape(N, 4)

    sn = source_node
    se = acc3[sn] / (den3[sn, 0] + 1e-16)
    se = (se + b3)[None, :]
    c1 = se @ Wm1[:OUT_DIM] + bm1[None, :]
    out = _make_mlp()(acc3, den3, b3, c1, Wm1[OUT_DIM:], Wm2, bm2, Wm3, bm3)
    return out[:, 0]


# speedup vs baseline: 64.5614x; 1.8496x over previous
import functools

import jax
import jax.numpy as jnp
from jax import lax
from jax.experimental import pallas as pl
from jax.experimental.pallas import tpu as pltpu
from jax.experimental.pallas import tpu_sc as plsc

N = 50000
E = 800000
HEADS = 4
HID = 16
OUT_DIM = 32
BN_SCALE = float(1.0 / (1.0 + 1e-5) ** 0.5)

R = 2000
NBLK = N // R


def _leaky(v):
    return jnp.where(v > 0, v, 0.2 * v)


_GDN = lax.GatherDimensionNumbers(offset_dims=(), collapsed_slice_dims=(0,),
                                  start_index_map=(0,))


def _vgather(v, idx):
    return lax.gather(v, idx[:, None], _GDN, (1,),
                      mode=lax.GatherScatterMode.PROMISE_IN_BOUNDS)


def _bcast_lane(v, lane_id):
    return _vgather(v, jnp.full((16,), lane_id, jnp.int32))


def _stage_pre_body(acc_in, den_in, W_ref, as_ref, ad_ref, b_ref, g_ref,
                    be_ref, s_ref, d_ref, ia_ref, id_ref, *, first, heads,
                    chid, pheads, pchid):
    hw = heads * chid
    if first:
        t = acc_in[...]
    else:
        phw = pheads * pchid
        agg = acc_in[...].reshape(R, pheads, pchid)
        den = den_in[...][:, :pheads].reshape(R, pheads, 1)
        z = (agg / (den + 1e-16)).reshape(R, phw) + b_ref[...]
        t = jax.nn.relu(g_ref[...] * z * BN_SCALE + be_ref[...])
    h = jnp.dot(t, W_ref[...], preferred_element_type=jnp.float32)
    hr = h.reshape(R, heads, chid)
    es = (hr * as_ref[...][None]).sum(-1)
    ed = (hr * ad_ref[...][None]).sum(-1)
    spad = s_ref.shape[1] - hw - heads
    s_ref[...] = jnp.concatenate(
        [h, es, jnp.zeros((R, spad), jnp.float32)], axis=1)
    d_ref[...] = jnp.concatenate(
        [ed, jnp.zeros((R, 16 - heads), jnp.float32)], axis=1)
    w = jnp.exp(_leaky(es + ed))
    ia_ref[...] = (hr * w[..., None]).reshape(R, hw)
    id_ref[...] = jnp.concatenate(
        [w, jnp.zeros((R, 4 - heads), jnp.float32)], axis=1) \
        if heads < 4 else w


def _make_stage_pre(in_w, heads, chid, s_w, first, pheads=HEADS, pchid=HID):
    body = functools.partial(_stage_pre_body, first=first, heads=heads,
                             chid=chid, pheads=pheads, pchid=pchid)
    hw = heads * chid
    w_in = in_w if first else pheads * pchid
    full = lambda *shape: pl.BlockSpec(shape, lambda i: (0,) * len(shape))
    return pl.pallas_call(
        body,
        grid=(NBLK,),
        in_specs=[
            pl.BlockSpec((R, in_w), lambda i: (i, 0)),
            pl.BlockSpec((R, 4), lambda i: (i, 0)),
            full(w_in, hw),
            full(heads, chid),
            full(heads, chid),
            full(pheads * pchid),
            full(pheads * pchid),
            full(pheads * pchid),
        ],
        out_specs=[
            pl.BlockSpec((R, s_w), lambda i: (i, 0)),
            pl.BlockSpec((R, 16), lambda i: (i, 0)),
            pl.BlockSpec((R, hw), lambda i: (i, 0)),
            pl.BlockSpec((R, 4), lambda i: (i, 0)),
        ],
        out_shape=[
            jax.ShapeDtypeStruct((N, s_w), jnp.float32),
            jax.ShapeDtypeStruct((N, 16), jnp.float32),
            jax.ShapeDtypeStruct((N, hw), jnp.float32),
            jax.ShapeDtypeStruct((N, 4), jnp.float32),
        ],
    )


def _mlp_body(acc_ref, den_ref, b3_ref, c1_ref, Wm1b_ref, Wm2_ref, bm2_ref,
              Wm3_ref, bm3_ref, out_ref):
    den = den_ref[...][:, :1]
    emb = acc_ref[...] / (den + 1e-16) + b3_ref[...]
    z = jax.nn.relu(jnp.dot(emb, Wm1b_ref[...],
                            preferred_element_type=jnp.float32) + c1_ref[...])
    z = jax.nn.relu(jnp.dot(z, Wm2_ref[...],
                            preferred_element_type=jnp.float32) + bm2_ref[...])
    out_ref[...] = jax.nn.sigmoid(
        jnp.dot(z, Wm3_ref[...], preferred_element_type=jnp.float32)
        + bm3_ref[...])


def _make_mlp():
    full = lambda *shape: pl.BlockSpec(shape, lambda i: (0,) * len(shape))
    return pl.pallas_call(
        _mlp_body,
        grid=(NBLK,),
        in_specs=[
            pl.BlockSpec((R, OUT_DIM), lambda i: (i, 0)),
            pl.BlockSpec((R, 4), lambda i: (i, 0)),
            full(OUT_DIM),
            full(1, 64),
            full(OUT_DIM, 64),
            full(64, 32),
            full(32),
            full(32, 1),
            full(1),
        ],
        out_specs=pl.BlockSpec((R, 1), lambda i: (i, 0)),
        out_shape=jax.ShapeDtypeStruct((N, 1), jnp.float32),
    )


NH = N // 2
B = 128
NB = 391
CT = B * NB
EP = 16 * CT
ACH = NH // 8


def _make_edge_sc(heads, chid, s_w):
    hw = heads * chid
    mesh = plsc.VectorSubcoreMesh(core_axis_name="c", subcore_axis_name="s")

    @functools.partial(
        pl.kernel, mesh=mesh,
        compiler_params=pltpu.CompilerParams(use_tc_tiling_on_sc=False),
        out_type=[
            jax.ShapeDtypeStruct((N, hw), jnp.float32),
            jax.ShapeDtypeStruct((N // 4, 16), jnp.float32),
        ],
        scratch_types=[
            pltpu.VMEM((B,), jnp.int32),
            pltpu.VMEM((B,), jnp.int32),
            pltpu.VMEM((B,), jnp.int32),
            pltpu.VMEM((B,), jnp.int32),
            pltpu.VMEM((B,), jnp.int32),
            pltpu.VMEM((B,), jnp.int32),
            pltpu.VMEM((B, s_w), jnp.float32),
            pltpu.VMEM((B, 16), jnp.float32),
            pltpu.VMEM((B, hw), jnp.float32),
            pltpu.VMEM((B, 16), jnp.float32),
            pltpu.VMEM_SHARED((NH + 8, hw), jnp.float32),
            pltpu.VMEM_SHARED((NH // 4 + 8, 16), jnp.float32),
            pltpu.SemaphoreType.DMA,
            pltpu.SemaphoreType.DMA,
        ],
    )
    def edge_kernel(S, Dt, IA, ID, srcp, dstp, accout, denout,
                    sidx, didx, gd, lidx, l4, lq, srows, drows, rrows, dnrows,
                    acc, dacc, sem1, sem2):
        cid = lax.axis_index("c")
        sid = lax.axis_index("s")
        lane = lax.iota(jnp.int32, 16)
        lanem4 = lane & 3
        lanegrp = lane >> 2
        headm = lanem4 < heads

        @pl.when(sid < 8)
        def _init_acc():
            pltpu.sync_copy(IA.at[pl.ds(cid * NH + sid * ACH, ACH)],
                            acc.at[pl.ds(sid * ACH, ACH)])

        @pl.when(sid == 8)
        def _init_den():
            pltpu.sync_copy(ID.at[pl.ds(cid * (NH // 4), NH // 4)],
                            dacc.at[pl.ds(0, NH // 4)])

        plsc.subcore_barrier()

        def batch_body(b, carry):
            base = sid * CT + b * B
            pltpu.sync_copy(srcp.at[pl.ds(base, B)], sidx)
            pltpu.sync_copy(dstp.at[pl.ds(base, B)], didx)
            for j in range(B // 16):
                sl = pl.ds(j * 16, 16)
                dv = didx[sl]
                gd[sl] = jnp.minimum(dv, N - 1)
                lv = dv - cid * NH
                ok = (lv >= 0) & (lv < NH)
                lv = jnp.where(ok, lv, NH)
                lidx[sl] = lv
                l4[sl] = lv >> 2
                lq[sl] = lv & 3
            cp1 = pltpu.async_copy(S.at[sidx], srows, sem1)
            cp2 = pltpu.async_copy(Dt.at[gd], drows, sem2)
            cp1.wait()
            cp2.wait()

            @plsc.parallel_loop(0, B, unroll=8)
            def edge_body(e):
                u = srows[e, pl.ds(hw, 16)] + drows[e, pl.ds(0, 16)]
                u = jnp.where(u > 0, u, 0.2 * u)
                w = jnp.exp(u)
                eb = (e >> 4) << 4
                q16 = lq[pl.ds(eb, 16)]
                qb = _vgather(q16, jnp.full((16,), 0, jnp.int32) + (e - eb))
                rep4 = _vgather(w, lanem4)
                dnrows[e, pl.ds(0, 16)] = jnp.where(
                    (lanegrp == qb) & headm, rep4, 0.0)
                for g in range(hw // 16):
                    hd = g // (chid // 16)
                    wb = _bcast_lane(w, hd)
                    rrows[e, pl.ds(g * 16, 16)] = (
                        wb * srows[e, pl.ds(g * 16, 16)])
            pltpu.sync_copy(rrows, acc.at[lidx], add=True)
            pltpu.sync_copy(dnrows, dacc.at[l4], add=True)
            return carry

        lax.fori_loop(0, NB, batch_body, 0)
        plsc.subcore_barrier()

        @pl.when(sid < 8)
        def _dump_acc():
            pltpu.sync_copy(acc.at[pl.ds(sid * ACH, ACH)],
                            accout.at[pl.ds(cid * NH + sid * ACH, ACH)])

        @pl.when(sid == 8)
        def _dump_den():
            pltpu.sync_copy(dacc.at[pl.ds(0, NH // 4)],
                            denout.at[pl.ds(cid * (NH // 4), NH // 4)])

    return edge_kernel


def kernel(x, edge_index, source_node, W1, a1s, a1d, b1, g1, be1, W2, a2s,
           a2d, b2, g2, be2, W3, a3s, a3d, b3, Wm1, bm1, Wm2, bm2, Wm3, bm3):
    src = edge_index[0].astype(jnp.int32)
    dst = edge_index[1].astype(jnp.int32)
    pad = EP - E
    srcp = jnp.concatenate([src, jnp.zeros((pad,), jnp.int32)])
    dstp = jnp.concatenate([dst, jnp.full((pad,), N, jnp.int32)])

    zeros64 = jnp.zeros((64,), jnp.float32)
    dummy_den = jnp.zeros((N, 4), jnp.float32)
    stage1 = _make_stage_pre(7, HEADS, HID, 80, True)
    S1, D1, IA1, ID1 = stage1(x, dummy_den, W1, a1s, a1d, zeros64, zeros64,
                              zeros64)
    edge44 = _make_edge_sc(HEADS, HID, 80)
    acc1, den1 = edge44(S1, D1, IA1, ID1.reshape(N // 4, 16), srcp, dstp)
    den1 = den1.reshape(N, 4)

    stage2 = _make_stage_pre(64, HEADS, HID, 80, False)
    S2, D2, IA2, ID2 = stage2(acc1, den1, W2, a2s, a2d, b1, g1, be1)
    acc2, den2 = edge44(S2, D2, IA2, ID2.reshape(N // 4, 16), srcp, dstp)
    den2 = den2.reshape(N, 4)

    stage3 = _make_stage_pre(64, 1, OUT_DIM, 48, False)
    S3, D3, IA3, ID3 = stage3(acc2, den2, W3, a3s, a3d, b2, g2, be2)
    edge13 = _make_edge_sc(1, OUT_DIM, 48)
    acc3, den3 = edge13(S3, D3, IA3, ID3.reshape(N // 4, 16), srcp, dstp)
    den3 = den3.reshape(N, 4)

    sn = source_node
    se = acc3[sn] / (den3[sn, 0] + 1e-16)
    se = (se + b3)[None, :]
    c1 = se @ Wm1[:OUT_DIM] + bm1[None, :]
    out = _make_mlp()(acc3, den3, b3, c1, Wm1[OUT_DIM:], Wm2, bm2, Wm3, bm3)
    return out[:, 0]


# speedup vs baseline: 79.6078x; 1.2331x over previous
import functools

import jax
import jax.numpy as jnp
from jax import lax
from jax.experimental import pallas as pl
from jax.experimental.pallas import tpu as pltpu
from jax.experimental.pallas import tpu_sc as plsc

N = 50000
E = 800000
HEADS = 4
HID = 16
OUT_DIM = 32
BN_SCALE = float(1.0 / (1.0 + 1e-5) ** 0.5)

R = 2000
NBLK = N // R


def _leaky(v):
    return jnp.where(v > 0, v, 0.2 * v)


_GDN = lax.GatherDimensionNumbers(offset_dims=(), collapsed_slice_dims=(0,),
                                  start_index_map=(0,))


def _vgather(v, idx):
    return lax.gather(v, idx[:, None], _GDN, (1,),
                      mode=lax.GatherScatterMode.PROMISE_IN_BOUNDS)


def _bcast_lane(v, lane_id):
    return _vgather(v, jnp.full((16,), lane_id, jnp.int32))


def _stage_pre_body(acc_in, den_in, W_ref, as_ref, ad_ref, b_ref, g_ref,
                    be_ref, s_ref, d_ref, ia_ref, id_ref, *, first, heads,
                    chid, pheads, pchid):
    hw = heads * chid
    if first:
        t = acc_in[...]
    else:
        phw = pheads * pchid
        agg = acc_in[...].reshape(R, pheads, pchid)
        den = den_in[...][:, :pheads].reshape(R, pheads, 1)
        z = (agg / (den + 1e-16)).reshape(R, phw) + b_ref[...]
        t = jax.nn.relu(g_ref[...] * z * BN_SCALE + be_ref[...])
    h = jnp.dot(t, W_ref[...], preferred_element_type=jnp.float32)
    hr = h.reshape(R, heads, chid)
    es = (hr * as_ref[...][None]).sum(-1)
    ed = (hr * ad_ref[...][None]).sum(-1)
    spad = s_ref.shape[1] - hw - heads
    s_ref[...] = jnp.concatenate(
        [h, es, jnp.zeros((R, spad), jnp.float32)], axis=1)
    d_ref[...] = jnp.concatenate(
        [ed, jnp.zeros((R, 16 - heads), jnp.float32)], axis=1)
    w = jnp.exp(_leaky(es + ed))
    ia_ref[...] = (hr * w[..., None]).reshape(R, hw)
    id_ref[...] = jnp.concatenate(
        [w, jnp.zeros((R, 4 - heads), jnp.float32)], axis=1) \
        if heads < 4 else w


def _make_stage_pre(in_w, heads, chid, s_w, first, pheads=HEADS, pchid=HID):
    body = functools.partial(_stage_pre_body, first=first, heads=heads,
                             chid=chid, pheads=pheads, pchid=pchid)
    hw = heads * chid
    w_in = in_w if first else pheads * pchid
    full = lambda *shape: pl.BlockSpec(shape, lambda i: (0,) * len(shape))
    return pl.pallas_call(
        body,
        grid=(NBLK,),
        in_specs=[
            pl.BlockSpec((R, in_w), lambda i: (i, 0)),
            pl.BlockSpec((R, 4), lambda i: (i, 0)),
            full(w_in, hw),
            full(heads, chid),
            full(heads, chid),
            full(pheads * pchid),
            full(pheads * pchid),
            full(pheads * pchid),
        ],
        out_specs=[
            pl.BlockSpec((R, s_w), lambda i: (i, 0)),
            pl.BlockSpec((R, 16), lambda i: (i, 0)),
            pl.BlockSpec((R, hw), lambda i: (i, 0)),
            pl.BlockSpec((R, 4), lambda i: (i, 0)),
        ],
        out_shape=[
            jax.ShapeDtypeStruct((N, s_w), jnp.float32),
            jax.ShapeDtypeStruct((N, 16), jnp.float32),
            jax.ShapeDtypeStruct((N, hw), jnp.float32),
            jax.ShapeDtypeStruct((N, 4), jnp.float32),
        ],
    )


def _mlp_body(acc_ref, den_ref, b3_ref, c1_ref, Wm1b_ref, Wm2_ref, bm2_ref,
              Wm3_ref, bm3_ref, out_ref):
    den = den_ref[...][:, :1]
    emb = acc_ref[...] / (den + 1e-16) + b3_ref[...]
    z = jax.nn.relu(jnp.dot(emb, Wm1b_ref[...],
                            preferred_element_type=jnp.float32) + c1_ref[...])
    z = jax.nn.relu(jnp.dot(z, Wm2_ref[...],
                            preferred_element_type=jnp.float32) + bm2_ref[...])
    out_ref[...] = jax.nn.sigmoid(
        jnp.dot(z, Wm3_ref[...], preferred_element_type=jnp.float32)
        + bm3_ref[...])


def _make_mlp():
    full = lambda *shape: pl.BlockSpec(shape, lambda i: (0,) * len(shape))
    return pl.pallas_call(
        _mlp_body,
        grid=(NBLK,),
        in_specs=[
            pl.BlockSpec((R, OUT_DIM), lambda i: (i, 0)),
            pl.BlockSpec((R, 4), lambda i: (i, 0)),
            full(OUT_DIM),
            full(1, 64),
            full(OUT_DIM, 64),
            full(64, 32),
            full(32),
            full(32, 1),
            full(1),
        ],
        out_specs=pl.BlockSpec((R, 1), lambda i: (i, 0)),
        out_shape=jax.ShapeDtypeStruct((N, 1), jnp.float32),
    )


NH = N // 2
B = 64
NB = 782
CT = B * NB
EP = 16 * CT
ACH = NH // 8


def _make_edge_sc(heads, chid, s_w):
    hw = heads * chid
    mesh = plsc.VectorSubcoreMesh(core_axis_name="c", subcore_axis_name="s")

    @functools.partial(
        pl.kernel, mesh=mesh,
        compiler_params=pltpu.CompilerParams(use_tc_tiling_on_sc=False),
        out_type=[
            jax.ShapeDtypeStruct((N, hw), jnp.float32),
            jax.ShapeDtypeStruct((N // 4, 16), jnp.float32),
        ],
        scratch_types=(
            [pltpu.VMEM((2, 2 * B), jnp.int32)]
            + [pltpu.VMEM((2, B), jnp.int32)] * 3
            + [pltpu.VMEM((2, B), jnp.int32)]
            + [pltpu.VMEM((2, B, s_w), jnp.float32)]
            + [pltpu.VMEM((2, B, 16), jnp.float32)]
            + [pltpu.VMEM((B, hw), jnp.float32)]
            + [pltpu.VMEM((B, 16), jnp.float32)]
            + [pltpu.VMEM_SHARED((NH + 8, hw), jnp.float32),
               pltpu.VMEM_SHARED((NH // 4 + 8, 16), jnp.float32)]
            + [pltpu.SemaphoreType.DMA] * 2
        ),
    )
    def edge_kernel(S, Dt, IA, ID, es_pk, accout, denout,
                    sd, gd, lidx, l4, lq, srows, drows, rrows, dnrows,
                    acc, dacc, gs, gt):
        cid = lax.axis_index("c")
        sid = lax.axis_index("s")
        lane = lax.iota(jnp.int32, 16)
        lanem4 = lane & 3
        lanegrp = lane >> 2
        headm = lanem4 < heads

        @pl.when(sid < 8)
        def _init_acc():
            pltpu.sync_copy(IA.at[pl.ds(cid * NH + sid * ACH, ACH)],
                            acc.at[pl.ds(sid * ACH, ACH)])

        @pl.when(sid == 8)
        def _init_den():
            pltpu.sync_copy(ID.at[pl.ds(cid * (NH // 4), NH // 4)],
                            dacc.at[pl.ds(0, NH // 4)])

        plsc.subcore_barrier()

        def loop_body(b, carry):
            p = b & 1

            @pl.when(b < NB)
            def _fetch():
                gb = sid * NB + b
                pltpu.sync_copy(es_pk.at[pl.ds(gb * 2 * B, 2 * B)], sd.at[p])
                for j in range(B // 16):
                    sl = pl.ds(j * 16, 16)
                    dv = sd[p, pl.ds(B + j * 16, 16)]
                    gd[p, sl] = jnp.minimum(dv, N - 1)
                    lv = dv - cid * NH
                    ok = (lv >= 0) & (lv < NH)
                    lv = jnp.where(ok, lv, NH)
                    lidx[p, sl] = lv
                    l4[p, sl] = lv >> 2
                    lq[p, sl] = lv & 3
                pltpu.async_copy(S.at[sd.at[p, pl.ds(0, B)]], srows.at[p],
                                 gs)
                pltpu.async_copy(Dt.at[gd.at[p]], drows.at[p], gt)

            @pl.when(b > 0)
            def _compute():
                q = 1 - p
                pltpu.make_async_copy(S.at[sd.at[q, pl.ds(0, B)]],
                                      srows.at[q], gs).wait()
                pltpu.make_async_copy(Dt.at[gd.at[q]], drows.at[q],
                                      gt).wait()

                @plsc.parallel_loop(0, B, unroll=8)
                def edge_body(e):
                    u = srows[q, e, pl.ds(hw, 16)] + drows[q, e, pl.ds(0, 16)]
                    u = jnp.where(u > 0, u, 0.2 * u)
                    w = jnp.exp(u)
                    eb = (e >> 4) << 4
                    q16 = lq[q, pl.ds(eb, 16)]
                    qb = _vgather(q16,
                                  jnp.full((16,), 0, jnp.int32) + (e - eb))
                    rep4 = _vgather(w, lanem4)
                    dnrows[e, pl.ds(0, 16)] = jnp.where(
                        (lanegrp == qb) & headm, rep4, 0.0)
                    for g in range(hw // 16):
                        hd = g // (chid // 16)
                        wb = _bcast_lane(w, hd)
                        rrows[e, pl.ds(g * 16, 16)] = (
                            wb * srows[q, e, pl.ds(g * 16, 16)])

                pltpu.sync_copy(rrows, acc.at[lidx.at[q]], add=True)
                pltpu.sync_copy(dnrows, dacc.at[l4.at[q]], add=True)

            return carry

        lax.fori_loop(0, NB + 1, loop_body, 0)
        plsc.subcore_barrier()

        @pl.when(sid < 8)
        def _dump_acc():
            pltpu.sync_copy(acc.at[pl.ds(sid * ACH, ACH)],
                            accout.at[pl.ds(cid * NH + sid * ACH, ACH)])

        @pl.when(sid == 8)
        def _dump_den():
            pltpu.sync_copy(dacc.at[pl.ds(0, NH // 4)],
                            denout.at[pl.ds(cid * (NH // 4), NH // 4)])

    return edge_kernel


def kernel(x, edge_index, source_node, W1, a1s, a1d, b1, g1, be1, W2, a2s,
           a2d, b2, g2, be2, W3, a3s, a3d, b3, Wm1, bm1, Wm2, bm2, Wm3, bm3):
    src = edge_index[0].astype(jnp.int32)
    dst = edge_index[1].astype(jnp.int32)
    pad = EP - E
    srcp = jnp.concatenate([src, jnp.zeros((pad,), jnp.int32)])
    dstp = jnp.concatenate([dst, jnp.full((pad,), N, jnp.int32)])
    es_pk = jnp.stack([srcp.reshape(-1, B), dstp.reshape(-1, B)],
                      axis=1).reshape(-1)

    zeros64 = jnp.zeros((64,), jnp.float32)
    dummy_den = jnp.zeros((N, 4), jnp.float32)
    stage1 = _make_stage_pre(7, HEADS, HID, 80, True)
    S1, D1, IA1, ID1 = stage1(x, dummy_den, W1, a1s, a1d, zeros64, zeros64,
                              zeros64)
    edge44 = _make_edge_sc(HEADS, HID, 80)
    acc1, den1 = edge44(S1, D1, IA1, ID1.reshape(N // 4, 16), es_pk)
    den1 = den1.reshape(N, 4)

    stage2 = _make_stage_pre(64, HEADS, HID, 80, False)
    S2, D2, IA2, ID2 = stage2(acc1, den1, W2, a2s, a2d, b1, g1, be1)
    acc2, den2 = edge44(S2, D2, IA2, ID2.reshape(N // 4, 16), es_pk)
    den2 = den2.reshape(N, 4)

    stage3 = _make_stage_pre(64, 1, OUT_DIM, 48, False)
    S3, D3, IA3, ID3 = stage3(acc2, den2, W3, a3s, a3d, b2, g2, be2)
    edge13 = _make_edge_sc(1, OUT_DIM, 48)
    acc3, den3 = edge13(S3, D3, IA3, ID3.reshape(N // 4, 16), es_pk)
    den3 = den3.reshape(N, 4)

    sn = source_node
    se = acc3[sn] / (den3[sn, 0] + 1e-16)
    se = (se + b3)[None, :]
    c1 = se @ Wm1[:OUT_DIM] + bm1[None, :]
    out = _make_mlp()(acc3, den3, b3, c1, Wm1[OUT_DIM:], Wm2, bm2, Wm3, bm3)
    return out[:, 0]
